# baseline (speedup 1.0000x reference)
import functools

import jax
import jax.numpy as jnp
from jax import lax
from jax.experimental import pallas as pl
from jax.experimental.pallas import tpu as pltpu
from jax.experimental.pallas import tpu_sc as plsc

ATOM_FDIM = 128
BOND_FDIM = 144
HIDDEN = 256
DEPTH = 3
N_ATOMS = 10000
N_BONDS = 320000
MAX_NB = 32
N_MOLS = 200

BT = 2048


def _mm0_body(x_ref, w_ref, inp_ref, msg_ref):
    acc = jnp.dot(x_ref[...], w_ref[...], preferred_element_type=jnp.float32)
    inp_ref[...] = acc
    msg_ref[...] = jnp.maximum(acc, 0.0)


def _mm0(f_bonds, W_i):
    grid = (N_BONDS // BT,)
    return pl.pallas_call(
        _mm0_body,
        grid=grid,
        in_specs=[
            pl.BlockSpec((BT, BOND_FDIM), lambda i: (i, 0)),
            pl.BlockSpec((BOND_FDIM, HIDDEN), lambda i: (0, 0)),
        ],
        out_specs=[
            pl.BlockSpec((BT, HIDDEN), lambda i: (i, 0)),
            pl.BlockSpec((BT, HIDDEN), lambda i: (i, 0)),
        ],
        out_shape=[
            jax.ShapeDtypeStruct((N_BONDS, HIDDEN), jnp.float32),
            jax.ShapeDtypeStruct((N_BONDS, HIDDEN), jnp.float32),
        ],
    )(f_bonds, W_i)


def _mmh_body(pre_ref, w_ref, inp_ref, msg_ref):
    acc = jnp.dot(pre_ref[...], w_ref[...], preferred_element_type=jnp.float32)
    msg_ref[...] = jnp.maximum(inp_ref[...] + acc, 0.0)


def _mmh(msg_pre, W_h, inp):
    grid = (N_BONDS // BT,)
    return pl.pallas_call(
        _mmh_body,
        grid=grid,
        in_specs=[
            pl.BlockSpec((BT, HIDDEN), lambda i: (i, 0)),
            pl.BlockSpec((HIDDEN, HIDDEN), lambda i: (0, 0)),
            pl.BlockSpec((BT, HIDDEN), lambda i: (i, 0)),
        ],
        out_specs=pl.BlockSpec((BT, HIDDEN), lambda i: (i, 0)),
        out_shape=jax.ShapeDtypeStruct((N_BONDS, HIDDEN), jnp.float32),
    )(msg_pre, W_h, inp)


AT = 2000
MOLP = 256


def _readout_body(fa_ref, am_ref, wo1_ref, wo2_ref, bo_ref, mid_ref,
                  sums_ref, cnts_ref):
    i = pl.program_id(0)
    h = jnp.dot(fa_ref[...], wo1_ref[...], preferred_element_type=jnp.float32)
    h = h + jnp.dot(am_ref[...], wo2_ref[...], preferred_element_type=jnp.float32)
    h = jnp.maximum(h + bo_ref[...], 0.0)
    ids = mid_ref[...]
    onehot = (ids == lax.broadcasted_iota(jnp.int32, (AT, MOLP), 1)).astype(jnp.float32)
    part_sums = jnp.dot(onehot.T, h, preferred_element_type=jnp.float32)
    part_cnts = jnp.sum(onehot, axis=0, keepdims=True)

    @pl.when(i == 0)
    def _init():
        sums_ref[...] = jnp.zeros_like(sums_ref)
        cnts_ref[...] = jnp.zeros_like(cnts_ref)

    sums_ref[...] += part_sums
    cnts_ref[...] += part_cnts


def _readout(f_atoms, a_message, W_o, b_o, mol_ids):
    W_o1 = W_o[:ATOM_FDIM]
    W_o2 = W_o[ATOM_FDIM:]
    grid = (N_ATOMS // AT,)
    sums, cnts = pl.pallas_call(
        _readout_body,
        grid=grid,
        in_specs=[
            pl.BlockSpec((AT, ATOM_FDIM), lambda i: (i, 0)),
            pl.BlockSpec((AT, HIDDEN), lambda i: (i, 0)),
            pl.BlockSpec((ATOM_FDIM, HIDDEN), lambda i: (0, 0)),
            pl.BlockSpec((HIDDEN, HIDDEN), lambda i: (0, 0)),
            pl.BlockSpec((1, HIDDEN), lambda i: (0, 0)),
            pl.BlockSpec((AT, 1), lambda i: (i, 0)),
        ],
        out_specs=[
            pl.BlockSpec((MOLP, HIDDEN), lambda i: (0, 0)),
            pl.BlockSpec((1, MOLP), lambda i: (0, 0)),
        ],
        out_shape=[
            jax.ShapeDtypeStruct((MOLP, HIDDEN), jnp.float32),
            jax.ShapeDtypeStruct((1, MOLP), jnp.float32),
        ],
    )(f_atoms, a_message, W_o1, W_o2, b_o.reshape(1, HIDDEN),
      mol_ids.reshape(N_ATOMS, 1))
    mol_vecs = sums[:N_MOLS] / jnp.maximum(cnts[0, :N_MOLS], 1.0)[:, None]
    return mol_vecs


NC, NS = 2, 16
NW = NC * NS
LANES = 16
CCH = HIDDEN // LANES

NA_PAD = 10240
NA_W = NA_PAD // NW
GA = 4
NGA = NA_W // GA

NB_W = N_BONDS // NW
GB = 40
NGB = NB_W // GB


def _sc_neisum(message, a2b_flat):
    mesh = plsc.VectorSubcoreMesh(core_axis_name="c", subcore_axis_name="s")

    @functools.partial(
        pl.kernel,
        out_type=jax.ShapeDtypeStruct((NA_PAD, HIDDEN), jnp.float32),
        mesh=mesh,
        scratch_types=[
            pltpu.VMEM((NA_W * MAX_NB,), jnp.int32),
            pltpu.VMEM((GA * MAX_NB, HIDDEN), jnp.float32),
            pltpu.VMEM((GA * MAX_NB, HIDDEN), jnp.float32),
            pltpu.VMEM((2 * GA, HIDDEN), jnp.float32),
            pltpu.SemaphoreType.DMA,
            pltpu.SemaphoreType.DMA,
        ],
    )
    def k(msg_hbm, a2b_hbm, out_hbm, idx_v, buf0, buf1, out_v, sem0, sem1):
        wid = lax.axis_index("s") * NC + lax.axis_index("c")
        ibase = wid * (NA_W * MAX_NB)
        pltpu.sync_copy(a2b_hbm.at[pl.ds(ibase, NA_W * MAX_NB)], idx_v)
        R = GA * MAX_NB
        pltpu.make_async_copy(
            msg_hbm.at[idx_v.at[pl.ds(0, R)]], buf0, sem0).start()

        @pl.loop(0, NGA // 2)
        def _outer(gh):
            for b in range(2):
                g = gh * 2 + b
                buf, sem = (buf0, sem0) if b == 0 else (buf1, sem1)
                nbuf, nsem = (buf1, sem1) if b == 0 else (buf0, sem0)

                @pl.when(g + 1 < NGA)
                def _fire():
                    pltpu.make_async_copy(
                        msg_hbm.at[idx_v.at[pl.ds((g + 1) * R, R)]],
                        nbuf, nsem).start()

                pltpu.make_async_copy(
                    msg_hbm.at[idx_v.at[pl.ds(g * R, R)]], buf, sem).wait()

                for j in range(GA):
                    @pl.loop(0, CCH)
                    def _cols(c):
                        col = pl.ds(c * LANES, LANES)
                        acc = buf[j * MAX_NB, col]
                        for kk in range(1, MAX_NB):
                            acc = acc + buf[j * MAX_NB + kk, col]
                        out_v[b * GA + j, col] = acc

                if b == 1:
                    pltpu.sync_copy(
                        out_v,
                        out_hbm.at[pl.ds(wid * NA_W + gh * (2 * GA), 2 * GA)])

    return k(message, a2b_flat)


def _sc_msgpre(a_message, message, b2a, b2revb):
    mesh = plsc.VectorSubcoreMesh(core_axis_name="c", subcore_axis_name="s")

    @functools.partial(
        pl.kernel,
        out_type=jax.ShapeDtypeStruct((N_BONDS, HIDDEN), jnp.float32),
        mesh=mesh,
        scratch_types=[
            pltpu.VMEM((NB_W,), jnp.int32),
            pltpu.VMEM((NB_W,), jnp.int32),
            pltpu.VMEM((GB, HIDDEN), jnp.float32),
            pltpu.VMEM((GB, HIDDEN), jnp.float32),
            pltpu.VMEM((GB, HIDDEN), jnp.float32),
            pltpu.VMEM((GB, HIDDEN), jnp.float32),
            pltpu.VMEM((GB, HIDDEN), jnp.float32),
            pltpu.SemaphoreType.DMA,
            pltpu.SemaphoreType.DMA,
            pltpu.SemaphoreType.DMA,
            pltpu.SemaphoreType.DMA,
        ],
    )
    def k(am_hbm, msg_hbm, b2a_hbm, b2revb_hbm, out_hbm,
          ia_v, ir_v, bufa0, bufa1, bufr0, bufr1, out_v,
          sa0, sa1, sr0, sr1):
        wid = lax.axis_index("s") * NC + lax.axis_index("c")
        base = wid * NB_W
        pltpu.sync_copy(b2a_hbm.at[pl.ds(base, NB_W)], ia_v)
        pltpu.sync_copy(b2revb_hbm.at[pl.ds(base, NB_W)], ir_v)
        pltpu.make_async_copy(
            am_hbm.at[ia_v.at[pl.ds(0, GB)]], bufa0, sa0).start()
        pltpu.make_async_copy(
            msg_hbm.at[ir_v.at[pl.ds(0, GB)]], bufr0, sr0).start()

        @pl.loop(0, NGB // 2)
        def _outer(gh):
            for b in range(2):
                g = gh * 2 + b
                bufa, sa = (bufa0, sa0) if b == 0 else (bufa1, sa1)
                bufr, sr = (bufr0, sr0) if b == 0 else (bufr1, sr1)
                nbufa, nsa = (bufa1, sa1) if b == 0 else (bufa0, sa0)
                nbufr, nsr = (bufr1, sr1) if b == 0 else (bufr0, sr0)

                @pl.when(g + 1 < NGB)
                def _fire():
                    pltpu.make_async_copy(
                        am_hbm.at[ia_v.at[pl.ds((g + 1) * GB, GB)]],
                        nbufa, nsa).start()
                    pltpu.make_async_copy(
                        msg_hbm.at[ir_v.at[pl.ds((g + 1) * GB, GB)]],
                        nbufr, nsr).start()

                pltpu.make_async_copy(
                    am_hbm.at[ia_v.at[pl.ds(g * GB, GB)]], bufa, sa).wait()
                pltpu.make_async_copy(
                    msg_hbm.at[ir_v.at[pl.ds(g * GB, GB)]], bufr, sr).wait()

                @pl.loop(0, CCH)
                def _cols(c):
                    col = pl.ds(c * LANES, LANES)
                    for j in range(GB):
                        out_v[j, col] = bufa[j, col] - bufr[j, col]

                pltpu.sync_copy(out_v, out_hbm.at[pl.ds(base + g * GB, GB)])

    return k(a_message, message, b2a, b2revb)


def kernel(f_atoms, f_bonds, W_i, W_h, W_o, b_o, a2b, b2a, b2revb, mol_ids):
    a2b_flat = jnp.pad(a2b.astype(jnp.int32),
                       ((0, NA_PAD - N_ATOMS), (0, 0))).reshape(-1)
    b2a = b2a.astype(jnp.int32)
    b2revb = b2revb.astype(jnp.int32)
    inp, message = _mm0(f_bonds, W_i)
    for _ in range(DEPTH - 1):
        a_message = _sc_neisum(message, a2b_flat)
        msg_pre = _sc_msgpre(a_message, message, b2a, b2revb)
        message = _mmh(msg_pre, W_h, inp)
    a_message = _sc_neisum(message, a2b_flat)
    return _readout(f_atoms, a_message[:N_ATOMS], W_o, b_o, mol_ids)


# speedup vs baseline: 1.1826x; 1.1826x over previous
import functools

import jax
import jax.numpy as jnp
from jax import lax
from jax.experimental import pallas as pl
from jax.experimental.pallas import tpu as pltpu
from jax.experimental.pallas import tpu_sc as plsc

ATOM_FDIM = 128
BOND_FDIM = 144
HIDDEN = 256
DEPTH = 3
N_ATOMS = 10000
N_BONDS = 320000
MAX_NB = 32
N_MOLS = 200

BF = jnp.bfloat16
HP = HIDDEN // 2
BT = 2048


def _pack(x_bf):
    lo = pltpu.bitcast(x_bf[:, :HP], jnp.uint16).astype(jnp.uint32)
    hi = pltpu.bitcast(x_bf[:, HP:], jnp.uint16).astype(jnp.uint32)
    return pltpu.bitcast(lo | (hi << 16), jnp.int32)


def _unpack(x_i32):
    u = pltpu.bitcast(x_i32, jnp.uint32)
    lo = pltpu.bitcast((u & 0xFFFF).astype(jnp.uint16), BF)
    hi = pltpu.bitcast((u >> 16).astype(jnp.uint16), BF)
    return jnp.concatenate([lo, hi], axis=1)


def _mm0_body(x_ref, w_ref, inp_ref, msg_ref):
    acc = jnp.dot(x_ref[...], w_ref[...], preferred_element_type=jnp.float32)
    inp_ref[...] = acc
    msg_ref[...] = _pack(jnp.maximum(acc, 0.0).astype(BF))


def _mm0(f_bonds_bf, W_i_bf):
    grid = (N_BONDS // BT,)
    return pl.pallas_call(
        _mm0_body,
        grid=grid,
        in_specs=[
            pl.BlockSpec((BT, BOND_FDIM), lambda i: (i, 0)),
            pl.BlockSpec((BOND_FDIM, HIDDEN), lambda i: (0, 0)),
        ],
        out_specs=[
            pl.BlockSpec((BT, HIDDEN), lambda i: (i, 0)),
            pl.BlockSpec((BT, HP), lambda i: (i, 0)),
        ],
        out_shape=[
            jax.ShapeDtypeStruct((N_BONDS, HIDDEN), jnp.float32),
            jax.ShapeDtypeStruct((N_BONDS, HP), jnp.int32),
        ],
    )(f_bonds_bf, W_i_bf)


def _mmh_body(pre_ref, w_ref, inp_ref, msg_ref):
    pre = _unpack(pre_ref[...])
    acc = jnp.dot(pre, w_ref[...], preferred_element_type=jnp.float32)
    msg_ref[...] = _pack(jnp.maximum(inp_ref[...] + acc, 0.0).astype(BF))


def _mmh(msg_pre, W_h_bf, inp):
    grid = (N_BONDS // BT,)
    return pl.pallas_call(
        _mmh_body,
        grid=grid,
        in_specs=[
            pl.BlockSpec((BT, HP), lambda i: (i, 0)),
            pl.BlockSpec((HIDDEN, HIDDEN), lambda i: (0, 0)),
            pl.BlockSpec((BT, HIDDEN), lambda i: (i, 0)),
        ],
        out_specs=pl.BlockSpec((BT, HP), lambda i: (i, 0)),
        out_shape=jax.ShapeDtypeStruct((N_BONDS, HP), jnp.int32),
    )(msg_pre, W_h_bf, inp)


AT = 2000
MOLP = 256


def _readout_body(fa_ref, am_ref, wo1_ref, wo2_ref, bo_ref, mid_ref,
                  sums_ref, cnts_ref):
    i = pl.program_id(0)
    h = jnp.dot(fa_ref[...], wo1_ref[...], preferred_element_type=jnp.float32)
    am = _unpack(am_ref[...]).astype(jnp.float32)
    h = h + jnp.dot(am, wo2_ref[...], preferred_element_type=jnp.float32)
    h = jnp.maximum(h + bo_ref[...], 0.0)
    ids = mid_ref[...]
    onehot = (ids == lax.broadcasted_iota(jnp.int32, (AT, MOLP), 1)).astype(jnp.float32)
    part_sums = jnp.dot(onehot.T, h, preferred_element_type=jnp.float32)
    part_cnts = jnp.sum(onehot, axis=0, keepdims=True)

    @pl.when(i == 0)
    def _init():
        sums_ref[...] = jnp.zeros_like(sums_ref)
        cnts_ref[...] = jnp.zeros_like(cnts_ref)

    sums_ref[...] += part_sums
    cnts_ref[...] += part_cnts


def _readout(f_atoms, a_message, W_o, b_o, mol_ids):
    W_o1 = W_o[:ATOM_FDIM]
    W_o2 = W_o[ATOM_FDIM:]
    grid = (N_ATOMS // AT,)
    sums, cnts = pl.pallas_call(
        _readout_body,
        grid=grid,
        in_specs=[
            pl.BlockSpec((AT, ATOM_FDIM), lambda i: (i, 0)),
            pl.BlockSpec((AT, HP), lambda i: (i, 0)),
            pl.BlockSpec((ATOM_FDIM, HIDDEN), lambda i: (0, 0)),
            pl.BlockSpec((HIDDEN, HIDDEN), lambda i: (0, 0)),
            pl.BlockSpec((1, HIDDEN), lambda i: (0, 0)),
            pl.BlockSpec((AT, 1), lambda i: (i, 0)),
        ],
        out_specs=[
            pl.BlockSpec((MOLP, HIDDEN), lambda i: (0, 0)),
            pl.BlockSpec((1, MOLP), lambda i: (0, 0)),
        ],
        out_shape=[
            jax.ShapeDtypeStruct((MOLP, HIDDEN), jnp.float32),
            jax.ShapeDtypeStruct((1, MOLP), jnp.float32),
        ],
    )(f_atoms, a_message, W_o1, W_o2, b_o.reshape(1, HIDDEN),
      mol_ids.reshape(N_ATOMS, 1))
    mol_vecs = sums[:N_MOLS] / jnp.maximum(cnts[0, :N_MOLS], 1.0)[:, None]
    return mol_vecs


MASKHI = -65536


def _lo_f32(w):
    return lax.bitcast_convert_type(lax.shift_left(w, 16), jnp.float32)


def _hi_f32(w):
    return lax.bitcast_convert_type(w & MASKHI, jnp.float32)


def _repack(lo_f, hi_f):
    lb = lax.bitcast_convert_type(lo_f, jnp.int32)
    hb = lax.bitcast_convert_type(hi_f, jnp.int32)
    lr = lax.shift_right_logical(
        lb + 0x7FFF + (lax.shift_right_logical(lb, 16) & 1), 16)
    hr = (hb + 0x7FFF + (lax.shift_right_logical(hb, 16) & 1)) & MASKHI
    return lr | hr


NC, NS = 2, 16
NW = NC * NS
WLAN = 32
CCH = HP // 16

NA_PAD = 10240
NA_W = NA_PAD // NW
GA = 4
NGA = NA_W // GA

NB_W = N_BONDS // NW
GB = 40
NGB = NB_W // GB


def _sc_neisum(message, a2b_flat):
    mesh = plsc.VectorSubcoreMesh(core_axis_name="c", subcore_axis_name="s")
    R = GA * MAX_NB

    @functools.partial(
        pl.kernel,
        out_type=jax.ShapeDtypeStruct((NA_PAD, HP), jnp.int32),
        mesh=mesh,
        scratch_types=[
            pltpu.VMEM((NA_W * MAX_NB,), jnp.int32),
            pltpu.VMEM((R, HP), jnp.int32),
            pltpu.VMEM((R, HP), jnp.int32),
            pltpu.VMEM((2 * GA, HP), jnp.int32),
            pltpu.VMEM((2 * GA, HP), jnp.int32),
            pltpu.SemaphoreType.DMA,
            pltpu.SemaphoreType.DMA,
            pltpu.SemaphoreType.DMA,
            pltpu.SemaphoreType.DMA,
        ],
    )
    def k(msg_hbm, a2b_hbm, out_hbm, idx_v, buf0, buf1, out0, out1,
          sem0, sem1, os0, os1):
        wid = lax.axis_index("s") * NC + lax.axis_index("c")
        ibase = pl.multiple_of(wid * (NA_W * MAX_NB), 8)
        pltpu.sync_copy(a2b_hbm.at[pl.ds(ibase, NA_W * MAX_NB)], idx_v)
        pltpu.make_async_copy(
            msg_hbm.at[idx_v.at[pl.ds(0, R)]], buf0, sem0).start()

        @pl.loop(0, NGA // 4)
        def _outer(oo):
            for f in range(2):
                fi = oo * 2 + f
                ov, osem = (out0, os0) if f == 0 else (out1, os1)

                @pl.when(fi >= 2)
                def _wait_store():
                    pltpu.make_async_copy(
                        ov, out_hbm.at[pl.ds(wid * NA_W + (fi - 2) * (2 * GA),
                                             2 * GA)], osem).wait()

                for b in range(2):
                    g = fi * 2 + b
                    buf, sem = (buf0, sem0) if b == 0 else (buf1, sem1)
                    nbuf, nsem = (buf1, sem1) if b == 0 else (buf0, sem0)

                    @pl.when(g + 1 < NGA)
                    def _fire():
                        pltpu.make_async_copy(
                            msg_hbm.at[idx_v.at[pl.ds(pl.multiple_of((g + 1) * R, 8), R)]],
                            nbuf, nsem).start()

                    pltpu.make_async_copy(
                        msg_hbm.at[idx_v.at[pl.ds(pl.multiple_of(g * R, 8), R)]], buf, sem).wait()

                    for j in range(GA):
                        @pl.loop(0, CCH)
                        def _cols(c):
                            col = pl.ds(c * 16, 16)
                            w = buf[j * MAX_NB, col]
                            acc_lo = _lo_f32(w)
                            acc_hi = _hi_f32(w)
                            for kk in range(1, MAX_NB):
                                w = buf[j * MAX_NB + kk, col]
                                acc_lo = acc_lo + _lo_f32(w)
                                acc_hi = acc_hi + _hi_f32(w)
                            ov[b * GA + j, col] = _repack(acc_lo, acc_hi)

                pltpu.make_async_copy(
                    ov, out_hbm.at[pl.ds(wid * NA_W + fi * (2 * GA), 2 * GA)],
                    osem).start()

        for f in range(2):
            fi = NGA // 2 - 2 + f
            ov, osem = (out0, os0) if fi % 2 == 0 else (out1, os1)
            pltpu.make_async_copy(
                ov, out_hbm.at[pl.ds(wid * NA_W + fi * (2 * GA), 2 * GA)],
                osem).wait()

    return k(message, a2b_flat)


def _sc_msgpre(a_message, message, b2a, b2revb):
    mesh = plsc.VectorSubcoreMesh(core_axis_name="c", subcore_axis_name="s")

    @functools.partial(
        pl.kernel,
        out_type=jax.ShapeDtypeStruct((N_BONDS, HP), jnp.int32),
        mesh=mesh,
        scratch_types=[
            pltpu.VMEM((NB_W,), jnp.int32),
            pltpu.VMEM((NB_W,), jnp.int32),
            pltpu.VMEM((GB, HP), jnp.int32),
            pltpu.VMEM((GB, HP), jnp.int32),
            pltpu.VMEM((GB, HP), jnp.int32),
            pltpu.VMEM((GB, HP), jnp.int32),
            pltpu.VMEM((GB, HP), jnp.int32),
            pltpu.VMEM((GB, HP), jnp.int32),
            pltpu.SemaphoreType.DMA,
            pltpu.SemaphoreType.DMA,
            pltpu.SemaphoreType.DMA,
            pltpu.SemaphoreType.DMA,
            pltpu.SemaphoreType.DMA,
            pltpu.SemaphoreType.DMA,
        ],
    )
    def k(am_hbm, msg_hbm, b2a_hbm, b2revb_hbm, out_hbm,
          ia_v, ir_v, bufa0, bufa1, bufr0, bufr1, out0, out1,
          sa0, sa1, sr0, sr1, os0, os1):
        wid = lax.axis_index("s") * NC + lax.axis_index("c")
        base = pl.multiple_of(wid * NB_W, 8)
        pltpu.sync_copy(b2a_hbm.at[pl.ds(base, NB_W)], ia_v)
        pltpu.sync_copy(b2revb_hbm.at[pl.ds(base, NB_W)], ir_v)
        pltpu.make_async_copy(
            am_hbm.at[ia_v.at[pl.ds(0, GB)]], bufa0, sa0).start()
        pltpu.make_async_copy(
            msg_hbm.at[ir_v.at[pl.ds(0, GB)]], bufr0, sr0).start()

        @pl.loop(0, NGB // 2)
        def _outer(gh):
            for b in range(2):
                g = gh * 2 + b
                bufa, sa = (bufa0, sa0) if b == 0 else (bufa1, sa1)
                bufr, sr = (bufr0, sr0) if b == 0 else (bufr1, sr1)
                nbufa, nsa = (bufa1, sa1) if b == 0 else (bufa0, sa0)
                nbufr, nsr = (bufr1, sr1) if b == 0 else (bufr0, sr0)
                ov, osem = (out0, os0) if b == 0 else (out1, os1)

                @pl.when(g + 1 < NGB)
                def _fire():
                    pltpu.make_async_copy(
                        am_hbm.at[ia_v.at[pl.ds(pl.multiple_of((g + 1) * GB, 8), GB)]],
                        nbufa, nsa).start()
                    pltpu.make_async_copy(
                        msg_hbm.at[ir_v.at[pl.ds(pl.multiple_of((g + 1) * GB, 8), GB)]],
                        nbufr, nsr).start()

                pltpu.make_async_copy(
                    am_hbm.at[ia_v.at[pl.ds(pl.multiple_of(g * GB, 8), GB)]], bufa, sa).wait()
                pltpu.make_async_copy(
                    msg_hbm.at[ir_v.at[pl.ds(pl.multiple_of(g * GB, 8), GB)]], bufr, sr).wait()

                @pl.when(g >= 2)
                def _wait_store():
                    pltpu.make_async_copy(
                        ov, out_hbm.at[pl.ds(base + (g - 2) * GB, GB)],
                        osem).wait()

                @pl.loop(0, CCH)
                def _cols(c):
                    col = pl.ds(c * 16, 16)
                    for j in range(GB):
                        wa = bufa[j, col]
                        wr = bufr[j, col]
                        lo = _lo_f32(wa) - _lo_f32(wr)
                        hi = _hi_f32(wa) - _hi_f32(wr)
                        ov[j, col] = _repack(lo, hi)

                pltpu.make_async_copy(
                    ov, out_hbm.at[pl.ds(base + g * GB, GB)], osem).start()

        for b in range(2):
            g = NGB - 2 + b
            ov, osem = (out0, os0) if b == 0 else (out1, os1)
            pltpu.make_async_copy(
                ov, out_hbm.at[pl.ds(base + g * GB, GB)], osem).wait()

    return k(a_message, message, b2a, b2revb)


def kernel(f_atoms, f_bonds, W_i, W_h, W_o, b_o, a2b, b2a, b2revb, mol_ids):
    a2b_flat = jnp.pad(a2b.astype(jnp.int32),
                       ((0, NA_PAD - N_ATOMS), (0, 0))).reshape(-1)
    b2a = b2a.astype(jnp.int32)
    b2revb = b2revb.astype(jnp.int32)
    inp, message = _mm0(f_bonds.astype(BF), W_i.astype(BF))
    W_h_bf = W_h.astype(BF)
    for _ in range(DEPTH - 1):
        a_message = _sc_neisum(message, a2b_flat)
        msg_pre = _sc_msgpre(a_message, message, b2a, b2revb)
        message = _mmh(msg_pre, W_h_bf, inp)
    a_message = _sc_neisum(message, a2b_flat)
    return _readout(f_atoms, a_message[:N_ATOMS], W_o, b_o, mol_ids)


# speedup vs baseline: 1.8401x; 1.5560x over previous
import functools

import jax
import jax.numpy as jnp
from jax import lax
from jax.experimental import pallas as pl
from jax.experimental.pallas import tpu as pltpu
from jax.experimental.pallas import tpu_sc as plsc

ATOM_FDIM = 128
BOND_FDIM = 144
HIDDEN = 256
DEPTH = 3
N_ATOMS = 10000
N_BONDS = 320000
MAX_NB = 32
N_MOLS = 200

BF = jnp.bfloat16
HP = HIDDEN // 2
BT = 2048


def _pack(x_bf):
    lo = pltpu.bitcast(x_bf[:, :HP], jnp.uint16).astype(jnp.uint32)
    hi = pltpu.bitcast(x_bf[:, HP:], jnp.uint16).astype(jnp.uint32)
    return pltpu.bitcast(lo | (hi << 16), jnp.int32)


def _unpack(x_i32):
    u = pltpu.bitcast(x_i32, jnp.uint32)
    lo = pltpu.bitcast((u & 0xFFFF).astype(jnp.uint16), BF)
    hi = pltpu.bitcast((u >> 16).astype(jnp.uint16), BF)
    return jnp.concatenate([lo, hi], axis=1)


def _mm0_body(x_ref, w_ref, inp_ref, msg_ref):
    acc = jnp.dot(x_ref[...], w_ref[...], preferred_element_type=jnp.float32)
    inp_ref[...] = acc
    msg_ref[...] = _pack(jnp.maximum(acc, 0.0).astype(BF))


def _mm0(f_bonds_bf, W_i_bf):
    grid = (N_BONDS // BT,)
    return pl.pallas_call(
        _mm0_body,
        grid=grid,
        in_specs=[
            pl.BlockSpec((BT, BOND_FDIM), lambda i: (i, 0)),
            pl.BlockSpec((BOND_FDIM, HIDDEN), lambda i: (0, 0)),
        ],
        out_specs=[
            pl.BlockSpec((BT, HIDDEN), lambda i: (i, 0)),
            pl.BlockSpec((BT, HP), lambda i: (i, 0)),
        ],
        out_shape=[
            jax.ShapeDtypeStruct((N_BONDS, HIDDEN), jnp.float32),
            jax.ShapeDtypeStruct((N_BONDS, HP), jnp.int32),
        ],
    )(f_bonds_bf, W_i_bf)


def _mmh_body(pre_ref, w_ref, inp_ref, msg_ref):
    pre = _unpack(pre_ref[...])
    acc = jnp.dot(pre, w_ref[...], preferred_element_type=jnp.float32)
    msg_ref[...] = _pack(jnp.maximum(inp_ref[...] + acc, 0.0).astype(BF))


def _mmh(msg_pre, W_h_bf, inp):
    grid = (N_BONDS // BT,)
    return pl.pallas_call(
        _mmh_body,
        grid=grid,
        in_specs=[
            pl.BlockSpec((BT, HP), lambda i: (i, 0)),
            pl.BlockSpec((HIDDEN, HIDDEN), lambda i: (0, 0)),
            pl.BlockSpec((BT, HIDDEN), lambda i: (i, 0)),
        ],
        out_specs=pl.BlockSpec((BT, HP), lambda i: (i, 0)),
        out_shape=jax.ShapeDtypeStruct((N_BONDS, HP), jnp.int32),
    )(msg_pre, W_h_bf, inp)


AT = 2000
MOLP = 256


def _readout_body(fa_ref, am_ref, wo1_ref, wo2_ref, bo_ref, mid_ref,
                  sums_ref, cnts_ref):
    i = pl.program_id(0)
    h = jnp.dot(fa_ref[...], wo1_ref[...], preferred_element_type=jnp.float32)
    am = _unpack(am_ref[...]).astype(jnp.float32)
    h = h + jnp.dot(am, wo2_ref[...], preferred_element_type=jnp.float32)
    h = jnp.maximum(h + bo_ref[...], 0.0)
    ids = mid_ref[...]
    onehot = (ids == lax.broadcasted_iota(jnp.int32, (AT, MOLP), 1)).astype(jnp.float32)
    part_sums = jnp.dot(onehot.T, h, preferred_element_type=jnp.float32)
    part_cnts = jnp.sum(onehot, axis=0, keepdims=True)

    @pl.when(i == 0)
    def _init():
        sums_ref[...] = jnp.zeros_like(sums_ref)
        cnts_ref[...] = jnp.zeros_like(cnts_ref)

    sums_ref[...] += part_sums
    cnts_ref[...] += part_cnts


def _readout(f_atoms, a_message, W_o, b_o, mol_ids):
    W_o1 = W_o[:ATOM_FDIM]
    W_o2 = W_o[ATOM_FDIM:]
    grid = (N_ATOMS // AT,)
    sums, cnts = pl.pallas_call(
        _readout_body,
        grid=grid,
        in_specs=[
            pl.BlockSpec((AT, ATOM_FDIM), lambda i: (i, 0)),
            pl.BlockSpec((AT, HP), lambda i: (i, 0)),
            pl.BlockSpec((ATOM_FDIM, HIDDEN), lambda i: (0, 0)),
            pl.BlockSpec((HIDDEN, HIDDEN), lambda i: (0, 0)),
            pl.BlockSpec((1, HIDDEN), lambda i: (0, 0)),
            pl.BlockSpec((AT, 1), lambda i: (i, 0)),
        ],
        out_specs=[
            pl.BlockSpec((MOLP, HIDDEN), lambda i: (0, 0)),
            pl.BlockSpec((1, MOLP), lambda i: (0, 0)),
        ],
        out_shape=[
            jax.ShapeDtypeStruct((MOLP, HIDDEN), jnp.float32),
            jax.ShapeDtypeStruct((1, MOLP), jnp.float32),
        ],
    )(f_atoms, a_message, W_o1, W_o2, b_o.reshape(1, HIDDEN),
      mol_ids.reshape(N_ATOMS, 1))
    mol_vecs = sums[:N_MOLS] / jnp.maximum(cnts[0, :N_MOLS], 1.0)[:, None]
    return mol_vecs


MASKHI = -65536


def _lo_f32(w):
    return lax.bitcast_convert_type(lax.shift_left(w, 16), jnp.float32)


def _hi_f32(w):
    return lax.bitcast_convert_type(w & MASKHI, jnp.float32)


def _repack(lo_f, hi_f):
    lb = lax.bitcast_convert_type(lo_f, jnp.int32)
    hb = lax.bitcast_convert_type(hi_f, jnp.int32)
    lr = lax.shift_right_logical(lb + 0x8000, 16)
    hr = (hb + 0x8000) & MASKHI
    return lr | hr


NC, NS = 2, 16
NW = NC * NS
WLAN = 32
CCH = HP // 16

NA_PAD = 10240
NA_W = NA_PAD // NW
GA = 4
NGA = NA_W // GA

NB_W = N_BONDS // NW
GB = 128
NGB = NB_W // GB
GBT = NB_W - NGB * GB


def _sc_neisum(message, a2b_flat):
    mesh = plsc.VectorSubcoreMesh(core_axis_name="c", subcore_axis_name="s")
    R = GA * MAX_NB

    @functools.partial(
        pl.kernel,
        out_type=jax.ShapeDtypeStruct((NA_PAD, HP), jnp.int32),
        mesh=mesh,
        scratch_types=[
            pltpu.VMEM((NA_W * MAX_NB,), jnp.int32),
            pltpu.VMEM((R, HP), jnp.int32),
            pltpu.VMEM((R, HP), jnp.int32),
            pltpu.VMEM((4 * GA, HP), jnp.int32),
            pltpu.SemaphoreType.DMA,
            pltpu.SemaphoreType.DMA,
        ],
    )
    def k(msg_hbm, a2b_hbm, out_hbm, idx_v, buf0, buf1, ov, sem0, sem1):
        wid = lax.axis_index("s") * NC + lax.axis_index("c")
        FR = 4 * GA
        ibase = pl.multiple_of(wid * (NA_W * MAX_NB), 8)
        pltpu.sync_copy(a2b_hbm.at[pl.ds(ibase, NA_W * MAX_NB)], idx_v)
        pltpu.make_async_copy(
            msg_hbm.at[idx_v.at[pl.ds(0, R)]], buf0, sem0).start()

        @pl.loop(0, NGA // 4)
        def _outer(fi):
            for b in range(4):
                g = fi * 4 + b
                buf, sem = (buf0, sem0) if b % 2 == 0 else (buf1, sem1)
                nbuf, nsem = (buf1, sem1) if b % 2 == 0 else (buf0, sem0)

                @pl.when(g + 1 < NGA)
                def _fire():
                    pltpu.make_async_copy(
                        msg_hbm.at[idx_v.at[pl.ds(pl.multiple_of((g + 1) * R, 8), R)]],
                        nbuf, nsem).start()

                pltpu.make_async_copy(
                    msg_hbm.at[idx_v.at[pl.ds(pl.multiple_of(g * R, 8), R)]], buf, sem).wait()

                for j in range(GA):
                    @pl.loop(0, CCH)
                    def _cols(c):
                        col = pl.ds(c * 16, 16)
                        w = buf[j * MAX_NB, col]
                        acc_lo = _lo_f32(w)
                        acc_hi = _hi_f32(w)
                        for kk in range(1, MAX_NB):
                            w = buf[j * MAX_NB + kk, col]
                            acc_lo = acc_lo + _lo_f32(w)
                            acc_hi = acc_hi + _hi_f32(w)
                        ov[b * GA + j, col] = _repack(acc_lo, acc_hi)

            pltpu.sync_copy(
                ov, out_hbm.at[pl.ds(pl.multiple_of(wid * NA_W + fi * FR, 8),
                                     FR)])

    return k(message, a2b_flat)


def _sc_msgpre(a_message, message, b2a, b2revb):
    mesh = plsc.VectorSubcoreMesh(core_axis_name="c", subcore_axis_name="s")

    @functools.partial(
        pl.kernel,
        out_type=jax.ShapeDtypeStruct((N_BONDS, HP), jnp.int32),
        mesh=mesh,
        scratch_types=[
            pltpu.VMEM((NB_W,), jnp.int32),
            pltpu.VMEM((NB_W,), jnp.int32),
            pltpu.VMEM((GB, HP), jnp.int32),
            pltpu.VMEM((GB, HP), jnp.int32),
            pltpu.VMEM((GB, HP), jnp.int32),
            pltpu.VMEM((GB, HP), jnp.int32),
            pltpu.VMEM((GB, HP), jnp.int32),
            pltpu.VMEM((GB, HP), jnp.int32),
            pltpu.SemaphoreType.DMA,
            pltpu.SemaphoreType.DMA,
            pltpu.SemaphoreType.DMA,
            pltpu.SemaphoreType.DMA,
            pltpu.SemaphoreType.DMA,
            pltpu.SemaphoreType.DMA,
        ],
    )
    def k(am_hbm, msg_hbm, b2a_hbm, b2revb_hbm, out_hbm,
          ia_v, ir_v, bufa0, bufa1, bufr0, bufr1, out0, out1,
          sa0, sa1, sr0, sr1, os0, os1):
        wid = lax.axis_index("s") * NC + lax.axis_index("c")
        base = pl.multiple_of(wid * NB_W, 8)
        pltpu.sync_copy(b2a_hbm.at[pl.ds(base, NB_W)], ia_v)
        pltpu.sync_copy(b2revb_hbm.at[pl.ds(base, NB_W)], ir_v)
        pltpu.make_async_copy(
            am_hbm.at[ia_v.at[pl.ds(0, GB)]], bufa0, sa0).start()
        pltpu.make_async_copy(
            msg_hbm.at[ir_v.at[pl.ds(0, GB)]], bufr0, sr0).start()

        def compute(bufa, bufr, ov, n):
            @pl.loop(0, n, unroll=2)
            def _rows(j):
                for c in range(CCH):
                    col = pl.ds(c * 16, 16)
                    wa = bufa[j, col]
                    wr = bufr[j, col]
                    lo = _lo_f32(wa) - _lo_f32(wr)
                    hi = _hi_f32(wa) - _hi_f32(wr)
                    ov[j, col] = _repack(lo, hi)

        @pl.loop(0, NGB // 2)
        def _outer(gh):
            for b in range(2):
                g = gh * 2 + b
                bufa, sa = (bufa0, sa0) if b == 0 else (bufa1, sa1)
                bufr, sr = (bufr0, sr0) if b == 0 else (bufr1, sr1)
                nbufa, nsa = (bufa1, sa1) if b == 0 else (bufa0, sa0)
                nbufr, nsr = (bufr1, sr1) if b == 0 else (bufr0, sr0)
                ov, osem = (out0, os0) if b == 0 else (out1, os1)

                @pl.when(g + 1 < NGB)
                def _fire():
                    pltpu.make_async_copy(
                        am_hbm.at[ia_v.at[pl.ds(pl.multiple_of((g + 1) * GB, 8), GB)]],
                        nbufa, nsa).start()
                    pltpu.make_async_copy(
                        msg_hbm.at[ir_v.at[pl.ds(pl.multiple_of((g + 1) * GB, 8), GB)]],
                        nbufr, nsr).start()

                pltpu.make_async_copy(
                    am_hbm.at[ia_v.at[pl.ds(pl.multiple_of(g * GB, 8), GB)]], bufa, sa).wait()
                pltpu.make_async_copy(
                    msg_hbm.at[ir_v.at[pl.ds(pl.multiple_of(g * GB, 8), GB)]], bufr, sr).wait()

                @pl.when(g >= 2)
                def _wait_store():
                    pltpu.make_async_copy(
                        ov, out_hbm.at[pl.ds(base + (g - 2) * GB, GB)],
                        osem).wait()

                compute(bufa, bufr, ov, GB)

                pltpu.make_async_copy(
                    ov, out_hbm.at[pl.ds(base + g * GB, GB)], osem).start()

        for b in range(2):
            g = NGB - 2 + b
            ov, osem = (out0, os0) if b == 0 else (out1, os1)
            pltpu.make_async_copy(
                ov, out_hbm.at[pl.ds(base + g * GB, GB)], osem).wait()

        tbase = pl.multiple_of(NGB * GB, 8)
        pltpu.make_async_copy(
            am_hbm.at[ia_v.at[pl.ds(tbase, GBT)]],
            bufa0.at[pl.ds(0, GBT)], sa0).start()
        pltpu.make_async_copy(
            msg_hbm.at[ir_v.at[pl.ds(tbase, GBT)]],
            bufr0.at[pl.ds(0, GBT)], sr0).start()
        pltpu.make_async_copy(
            am_hbm.at[ia_v.at[pl.ds(tbase, GBT)]],
            bufa0.at[pl.ds(0, GBT)], sa0).wait()
        pltpu.make_async_copy(
            msg_hbm.at[ir_v.at[pl.ds(tbase, GBT)]],
            bufr0.at[pl.ds(0, GBT)], sr0).wait()
        compute(bufa0, bufr0, out0, GBT)
        pltpu.sync_copy(out0.at[pl.ds(0, GBT)],
                        out_hbm.at[pl.ds(base + NGB * GB, GBT)])

    return k(a_message, message, b2a, b2revb)


def kernel(f_atoms, f_bonds, W_i, W_h, W_o, b_o, a2b, b2a, b2revb, mol_ids):
    pad_idx = (jnp.arange((NA_PAD - N_ATOMS) * MAX_NB, dtype=jnp.int32)
               % N_BONDS).reshape(NA_PAD - N_ATOMS, MAX_NB)
    a2b_flat = jnp.concatenate([a2b.astype(jnp.int32), pad_idx], 0).reshape(-1)
    b2a = b2a.astype(jnp.int32)
    b2revb = b2revb.astype(jnp.int32)
    inp, message = _mm0(f_bonds.astype(BF), W_i.astype(BF))
    W_h_bf = W_h.astype(BF)
    for _ in range(DEPTH - 1):
        a_message = _sc_neisum(message, a2b_flat)
        msg_pre = _sc_msgpre(a_message, message, b2a, b2revb)
        message = _mmh(msg_pre, W_h_bf, inp)
    a_message = _sc_neisum(message, a2b_flat)
    return _readout(f_atoms, a_message[:N_ATOMS], W_o, b_o, mol_ids)


# speedup vs baseline: 1.9987x; 1.0862x over previous
import functools

import jax
import jax.numpy as jnp
from jax import lax
from jax.experimental import pallas as pl
from jax.experimental.pallas import tpu as pltpu
from jax.experimental.pallas import tpu_sc as plsc

ATOM_FDIM = 128
BOND_FDIM = 144
HIDDEN = 256
DEPTH = 3
N_ATOMS = 10000
N_BONDS = 320000
MAX_NB = 32
N_MOLS = 200

BF = jnp.bfloat16
HP = HIDDEN // 2
BT = 2000


def _pack(x_bf):
    lo = pltpu.bitcast(x_bf[:, :HP], jnp.uint16).astype(jnp.uint32)
    hi = pltpu.bitcast(x_bf[:, HP:], jnp.uint16).astype(jnp.uint32)
    return pltpu.bitcast(lo | (hi << 16), jnp.int32)


def _unpack(x_i32):
    u = pltpu.bitcast(x_i32, jnp.uint32)
    lo = pltpu.bitcast((u & 0xFFFF).astype(jnp.uint16), BF)
    hi = pltpu.bitcast((u >> 16).astype(jnp.uint16), BF)
    return jnp.concatenate([lo, hi], axis=1)


def _mm0_body(x_ref, w_ref, inp_ref, msg_ref):
    acc = jnp.dot(x_ref[...], w_ref[...], preferred_element_type=jnp.float32)
    inp_ref[...] = acc
    msg_ref[...] = _pack(jnp.maximum(acc, 0.0).astype(BF))


def _mm0(f_bonds_bf, W_i_bf):
    grid = (N_BONDS // BT,)
    return pl.pallas_call(
        _mm0_body,
        grid=grid,
        in_specs=[
            pl.BlockSpec((BT, BOND_FDIM), lambda i: (i, 0)),
            pl.BlockSpec((BOND_FDIM, HIDDEN), lambda i: (0, 0)),
        ],
        out_specs=[
            pl.BlockSpec((BT, HIDDEN), lambda i: (i, 0)),
            pl.BlockSpec((BT, HP), lambda i: (i, 0)),
        ],
        out_shape=[
            jax.ShapeDtypeStruct((N_BONDS, HIDDEN), jnp.float32),
            jax.ShapeDtypeStruct((N_BONDS, HP), jnp.int32),
        ],
    )(f_bonds_bf, W_i_bf)


def _mmh_body(dst_ref, pre_ref, w_ref, inp_ref, msg_ref):
    del dst_ref
    pre = _unpack(pre_ref[...])
    acc = jnp.dot(pre, w_ref[...], preferred_element_type=jnp.float32)
    msg_ref[...] = _pack(jnp.maximum(inp_ref[...] + acc, 0.0).astype(BF))


def _mmh_chunk(m_buf, msg_pre_half, W_h_bf, inp, off_blocks):
    grid = (msg_pre_half.shape[0] // BT,)
    return pl.pallas_call(
        _mmh_body,
        grid=grid,
        in_specs=[
            pl.BlockSpec(memory_space=pl.ANY),
            pl.BlockSpec((BT, HP), lambda i: (i, 0)),
            pl.BlockSpec((HIDDEN, HIDDEN), lambda i: (0, 0)),
            pl.BlockSpec((BT, HIDDEN), lambda i: (i + off_blocks, 0)),
        ],
        out_specs=pl.BlockSpec((BT, HP), lambda i: (i + off_blocks, 0)),
        out_shape=jax.ShapeDtypeStruct((N_BONDS, HP), jnp.int32),
        input_output_aliases={0: 0},
    )(m_buf, msg_pre_half, W_h_bf, inp)


AT = 2000
MOLP = 256


def _readout_body(fa_ref, am_ref, wo1_ref, wo2_ref, bo_ref, mid_ref,
                  sums_ref, cnts_ref):
    i = pl.program_id(0)
    h = jnp.dot(fa_ref[...], wo1_ref[...], preferred_element_type=jnp.float32)
    am = _unpack(am_ref[...]).astype(jnp.float32)
    h = h + jnp.dot(am, wo2_ref[...], preferred_element_type=jnp.float32)
    h = jnp.maximum(h + bo_ref[...], 0.0)
    ids = mid_ref[...]
    onehot = (ids == lax.broadcasted_iota(jnp.int32, (AT, MOLP), 1)).astype(jnp.float32)
    part_sums = jnp.dot(onehot.T, h, preferred_element_type=jnp.float32)
    part_cnts = jnp.sum(onehot, axis=0, keepdims=True)

    @pl.when(i == 0)
    def _init():
        sums_ref[...] = jnp.zeros_like(sums_ref)
        cnts_ref[...] = jnp.zeros_like(cnts_ref)

    sums_ref[...] += part_sums
    cnts_ref[...] += part_cnts


def _readout(f_atoms, a_message, W_o, b_o, mol_ids):
    W_o1 = W_o[:ATOM_FDIM]
    W_o2 = W_o[ATOM_FDIM:]
    grid = (N_ATOMS // AT,)
    sums, cnts = pl.pallas_call(
        _readout_body,
        grid=grid,
        in_specs=[
            pl.BlockSpec((AT, ATOM_FDIM), lambda i: (i, 0)),
            pl.BlockSpec((AT, HP), lambda i: (i, 0)),
            pl.BlockSpec((ATOM_FDIM, HIDDEN), lambda i: (0, 0)),
            pl.BlockSpec((HIDDEN, HIDDEN), lambda i: (0, 0)),
            pl.BlockSpec((1, HIDDEN), lambda i: (0, 0)),
            pl.BlockSpec((AT, 1), lambda i: (i, 0)),
        ],
        out_specs=[
            pl.BlockSpec((MOLP, HIDDEN), lambda i: (0, 0)),
            pl.BlockSpec((1, MOLP), lambda i: (0, 0)),
        ],
        out_shape=[
            jax.ShapeDtypeStruct((MOLP, HIDDEN), jnp.float32),
            jax.ShapeDtypeStruct((1, MOLP), jnp.float32),
        ],
    )(f_atoms, a_message, W_o1, W_o2, b_o.reshape(1, HIDDEN),
      mol_ids.reshape(N_ATOMS, 1))
    mol_vecs = sums[:N_MOLS] / jnp.maximum(cnts[0, :N_MOLS], 1.0)[:, None]
    return mol_vecs


MASKHI = -65536


def _lo_f32(w):
    return lax.bitcast_convert_type(lax.shift_left(w, 16), jnp.float32)


def _hi_f32(w):
    return lax.bitcast_convert_type(w & MASKHI, jnp.float32)


def _repack(lo_f, hi_f):
    lb = lax.bitcast_convert_type(lo_f, jnp.int32)
    hb = lax.bitcast_convert_type(hi_f, jnp.int32)
    lr = lax.shift_right_logical(lb + 0x8000, 16)
    hr = (hb + 0x8000) & MASKHI
    return lr | hr


NC, NS = 2, 16
NW = NC * NS
WLAN = 32
CCH = HP // 16

NA_PAD = 10240
NA_W = NA_PAD // NW
GA = 4
NGA = NA_W // GA

NB_W = N_BONDS // NW
GB = 128
NGB = NB_W // GB
GBT = NB_W - NGB * GB


def _sc_neisum(message, a2b_flat):
    mesh = plsc.VectorSubcoreMesh(core_axis_name="c", subcore_axis_name="s")
    R = GA * MAX_NB

    @functools.partial(
        pl.kernel,
        out_type=jax.ShapeDtypeStruct((NA_PAD, HP), jnp.int32),
        mesh=mesh,
        scratch_types=[
            pltpu.VMEM((NA_W * MAX_NB,), jnp.int32),
            pltpu.VMEM((R, HP), jnp.int32),
            pltpu.VMEM((R, HP), jnp.int32),
            pltpu.VMEM((4 * GA, HP), jnp.int32),
            pltpu.SemaphoreType.DMA,
            pltpu.SemaphoreType.DMA,
        ],
    )
    def k(msg_hbm, a2b_hbm, out_hbm, idx_v, buf0, buf1, ov, sem0, sem1):
        wid = lax.axis_index("s") * NC + lax.axis_index("c")
        FR = 4 * GA
        ibase = pl.multiple_of(wid * (NA_W * MAX_NB), 8)
        pltpu.sync_copy(a2b_hbm.at[pl.ds(ibase, NA_W * MAX_NB)], idx_v)
        pltpu.make_async_copy(
            msg_hbm.at[idx_v.at[pl.ds(0, R)]], buf0, sem0).start()

        @pl.loop(0, NGA // 4)
        def _outer(fi):
            for b in range(4):
                g = fi * 4 + b
                buf, sem = (buf0, sem0) if b % 2 == 0 else (buf1, sem1)
                nbuf, nsem = (buf1, sem1) if b % 2 == 0 else (buf0, sem0)

                @pl.when(g + 1 < NGA)
                def _fire():
                    pltpu.make_async_copy(
                        msg_hbm.at[idx_v.at[pl.ds(pl.multiple_of((g + 1) * R, 8), R)]],
                        nbuf, nsem).start()

                pltpu.make_async_copy(
                    msg_hbm.at[idx_v.at[pl.ds(pl.multiple_of(g * R, 8), R)]], buf, sem).wait()

                for j in range(GA):
                    @pl.loop(0, CCH)
                    def _cols(c):
                        col = pl.ds(c * 16, 16)
                        w = buf[j * MAX_NB, col]
                        acc_lo = _lo_f32(w)
                        acc_hi = _hi_f32(w)
                        for kk in range(1, MAX_NB):
                            w = buf[j * MAX_NB + kk, col]
                            acc_lo = acc_lo + _lo_f32(w)
                            acc_hi = acc_hi + _hi_f32(w)
                        ov[b * GA + j, col] = _repack(acc_lo, acc_hi)

            pltpu.sync_copy(
                ov, out_hbm.at[pl.ds(pl.multiple_of(wid * NA_W + fi * FR, 8),
                                     FR)])

    return k(message, a2b_flat)


def _sc_msgpre(a_message, message, b2a, b2revb):
    mesh = plsc.VectorSubcoreMesh(core_axis_name="c", subcore_axis_name="s")
    nb = b2a.shape[0]
    nbw = nb // NW
    ngb = nbw // GB
    gbt = nbw - ngb * GB

    @functools.partial(
        pl.kernel,
        out_type=jax.ShapeDtypeStruct((nb, HP), jnp.int32),
        mesh=mesh,
        scratch_types=[
            pltpu.VMEM((nbw,), jnp.int32),
            pltpu.VMEM((nbw,), jnp.int32),
            pltpu.VMEM((GB, HP), jnp.int32),
            pltpu.VMEM((GB, HP), jnp.int32),
            pltpu.VMEM((GB, HP), jnp.int32),
            pltpu.VMEM((GB, HP), jnp.int32),
            pltpu.VMEM((GB, HP), jnp.int32),
            pltpu.VMEM((GB, HP), jnp.int32),
            pltpu.SemaphoreType.DMA,
            pltpu.SemaphoreType.DMA,
            pltpu.SemaphoreType.DMA,
            pltpu.SemaphoreType.DMA,
            pltpu.SemaphoreType.DMA,
            pltpu.SemaphoreType.DMA,
        ],
    )
    def k(am_hbm, msg_hbm, b2a_hbm, b2revb_hbm, out_hbm,
          ia_v, ir_v, bufa0, bufa1, bufr0, bufr1, out0, out1,
          sa0, sa1, sr0, sr1, os0, os1):
        wid = lax.axis_index("s") * NC + lax.axis_index("c")
        base = pl.multiple_of(wid * nbw, 8)
        pltpu.sync_copy(b2a_hbm.at[pl.ds(base, nbw)], ia_v)
        pltpu.sync_copy(b2revb_hbm.at[pl.ds(base, nbw)], ir_v)
        pltpu.make_async_copy(
            am_hbm.at[ia_v.at[pl.ds(0, GB)]], bufa0, sa0).start()
        pltpu.make_async_copy(
            msg_hbm.at[ir_v.at[pl.ds(0, GB)]], bufr0, sr0).start()

        def compute(bufa, bufr, ov, n):
            @pl.loop(0, n, unroll=2)
            def _rows(j):
                for c in range(CCH):
                    col = pl.ds(c * 16, 16)
                    wa = bufa[j, col]
                    wr = bufr[j, col]
                    lo = _lo_f32(wa) - _lo_f32(wr)
                    hi = _hi_f32(wa) - _hi_f32(wr)
                    ov[j, col] = _repack(lo, hi)

        bufs = [(bufa0, sa0, bufr0, sr0, out0, os0),
                (bufa1, sa1, bufr1, sr1, out1, os1)]

        def do_group(g, b):
            bufa, sa, bufr, sr, ov, osem = bufs[b]
            nbufa, nsa, nbufr, nsr, _, _ = bufs[1 - b]

            @pl.when(g + 1 < ngb)
            def _fire():
                pltpu.make_async_copy(
                    am_hbm.at[ia_v.at[pl.ds(pl.multiple_of((g + 1) * GB, 8), GB)]],
                    nbufa, nsa).start()
                pltpu.make_async_copy(
                    msg_hbm.at[ir_v.at[pl.ds(pl.multiple_of((g + 1) * GB, 8), GB)]],
                    nbufr, nsr).start()

            pltpu.make_async_copy(
                am_hbm.at[ia_v.at[pl.ds(pl.multiple_of(g * GB, 8), GB)]], bufa, sa).wait()
            pltpu.make_async_copy(
                msg_hbm.at[ir_v.at[pl.ds(pl.multiple_of(g * GB, 8), GB)]], bufr, sr).wait()

            @pl.when(g >= 2)
            def _wait_store():
                pltpu.make_async_copy(
                    ov, out_hbm.at[pl.ds(base + (g - 2) * GB, GB)],
                    osem).wait()

            compute(bufa, bufr, ov, GB)

            pltpu.make_async_copy(
                ov, out_hbm.at[pl.ds(base + g * GB, GB)], osem).start()

        @pl.loop(0, ngb // 2)
        def _outer(gh):
            for b in range(2):
                do_group(gh * 2 + b, b)

        if ngb % 2 == 1:
            do_group(ngb - 1, 0)

        for g in (ngb - 2, ngb - 1):
            _, _, _, _, ov, osem = bufs[g % 2]
            pltpu.make_async_copy(
                ov, out_hbm.at[pl.ds(base + g * GB, GB)], osem).wait()

        if gbt > 0:
            tp = 1 if ngb % 2 == 1 else 0
            tbufa, tsa, tbufr, tsr, tov, _ = bufs[tp]
            tbase = pl.multiple_of(ngb * GB, 8)
            pltpu.make_async_copy(
                am_hbm.at[ia_v.at[pl.ds(tbase, gbt)]],
                tbufa.at[pl.ds(0, gbt)], tsa).start()
            pltpu.make_async_copy(
                msg_hbm.at[ir_v.at[pl.ds(tbase, gbt)]],
                tbufr.at[pl.ds(0, gbt)], tsr).start()
            pltpu.make_async_copy(
                am_hbm.at[ia_v.at[pl.ds(tbase, gbt)]],
                tbufa.at[pl.ds(0, gbt)], tsa).wait()
            pltpu.make_async_copy(
                msg_hbm.at[ir_v.at[pl.ds(tbase, gbt)]],
                tbufr.at[pl.ds(0, gbt)], tsr).wait()
            compute(tbufa, tbufr, tov, gbt)
            pltpu.sync_copy(tov.at[pl.ds(0, gbt)],
                            out_hbm.at[pl.ds(base + ngb * GB, gbt)])

    return k(a_message, message, b2a, b2revb)


def kernel(f_atoms, f_bonds, W_i, W_h, W_o, b_o, a2b, b2a, b2revb, mol_ids):
    pad_idx = (jnp.arange((NA_PAD - N_ATOMS) * MAX_NB, dtype=jnp.int32)
               % N_BONDS).reshape(NA_PAD - N_ATOMS, MAX_NB)
    a2b_flat = jnp.concatenate([a2b.astype(jnp.int32), pad_idx], 0).reshape(-1)
    b2a = b2a.astype(jnp.int32)
    b2revb = b2revb.astype(jnp.int32)
    inp, message = _mm0(f_bonds.astype(BF), W_i.astype(BF))
    W_h_bf = W_h.astype(BF)
    H2 = N_BONDS // 2
    for _ in range(DEPTH - 1):
        a_message = _sc_neisum(message, a2b_flat)
        pre0 = _sc_msgpre(a_message, message, b2a[:H2], b2revb[:H2])
        pre1 = _sc_msgpre(a_message, message, b2a[H2:], b2revb[H2:])
        m = jnp.zeros((N_BONDS, HP), jnp.int32)
        m = _mmh_chunk(m, pre0, W_h_bf, inp, 0)
        message = _mmh_chunk(m, pre1, W_h_bf, inp, H2 // BT)
    a_message = _sc_neisum(message, a2b_flat)
    return _readout(f_atoms, a_message[:N_ATOMS], W_o, b_o, mol_ids)
